# Initial kernel scaffold; baseline (speedup 1.0000x reference)
import jax
import jax.numpy as jnp
from jax.experimental import pallas as pl


def kernel(x, edge_index, Wl1, bl1, Wr1, Wl2, bl2, Wr2, Wout, bout):
    raise NotImplementedError("write your pallas kernel here")


# speedup vs baseline: 7.2091x; 7.2091x over previous
import jax
import jax.numpy as jnp
from jax import lax
from jax.experimental import pallas as pl
from jax.experimental.pallas import tpu as pltpu
from jax.experimental.pallas import tpu_sc as plsc

N = 10000
E = 320000
D = 128
H = 128
C = 64

NC = 2
NS = 16
NW = NC * NS
EPT = E // NW
CK = 125
NCHUNK = EPT // CK
NPAD = 10240
NPT = NPAD // NS
ZB = 128
LANES = 16


def _sc_agg_body(h_hbm, src_hbm, dst_hbm, sums_hbm,
                 src_v, dst_v, rows_v, acc_sh, gsem):
    c = lax.axis_index("c")
    s = lax.axis_index("s")
    wid = c * NS + s

    def zero_rows(k, _):
        r = k // (D // LANES)
        col = (k % (D // LANES)) * LANES
        rows_v[r, pl.ds(col, LANES)] = jnp.zeros((LANES,), jnp.float32)
        return _
    lax.fori_loop(0, ZB * (D // LANES), zero_rows, None)
    for k in range(NPT // ZB):
        pltpu.sync_copy(rows_v, acc_sh.at[pl.ds(s * NPT + k * ZB, ZB), :])

    pltpu.sync_copy(src_hbm.at[wid], src_v)
    pltpu.sync_copy(dst_hbm.at[wid], dst_v)

    plsc.subcore_barrier()

    def step(j, _):
        pltpu.async_copy(h_hbm.at[src_v.at[j]], rows_v.at[pl.ds(0, CK), :],
                         gsem).wait()
        pltpu.sync_copy(rows_v.at[pl.ds(0, CK), :],
                        acc_sh.at[dst_v.at[j]], add=True)
        return _
    lax.fori_loop(0, NCHUNK, step, None)

    plsc.subcore_barrier()

    pltpu.sync_copy(acc_sh.at[pl.ds(s * NPT, NPT), :],
                    sums_hbm.at[c, pl.ds(s * NPT, NPT), :])


def _sc_cnt_body(dst_hbm, cnt_hbm, dst_v, ones_v, cnt_sh, gsem):
    c = lax.axis_index("c")
    s = lax.axis_index("s")
    wid = c * NS + s

    def store_const(val, k, _):
        r = k // (D // LANES)
        col = (k % (D // LANES)) * LANES
        ones_v[r, pl.ds(col, LANES)] = jnp.full((LANES,), val, jnp.float32)
        return _

    lax.fori_loop(0, ZB * (D // LANES),
                  lambda k, _: store_const(0.0, k, _), None)
    for k in range(NPT // ZB):
        pltpu.sync_copy(ones_v, cnt_sh.at[pl.ds(s * NPT + k * ZB, ZB), :])

    lax.fori_loop(0, ZB * (D // LANES),
                  lambda k, _: store_const(1.0, k, _), None)

    pltpu.sync_copy(dst_hbm.at[wid], dst_v)

    plsc.subcore_barrier()

    def step(j, _):
        pltpu.sync_copy(ones_v.at[pl.ds(0, CK), :],
                        cnt_sh.at[dst_v.at[j]], add=True)
        return _
    lax.fori_loop(0, NCHUNK, step, None)

    plsc.subcore_barrier()

    pltpu.sync_copy(cnt_sh.at[pl.ds(s * NPT, NPT), :],
                    cnt_hbm.at[c, pl.ds(s * NPT, NPT), :])


_sc_mesh = plsc.VectorSubcoreMesh(core_axis_name="c", subcore_axis_name="s")

_sc_agg = pl.kernel(
    _sc_agg_body,
    out_type=(jax.ShapeDtypeStruct((NC, NPAD, D), jnp.float32),),
    mesh=_sc_mesh,
    scratch_types=[
        pltpu.VMEM((NCHUNK, CK), jnp.int32),
        pltpu.VMEM((NCHUNK, CK), jnp.int32),
        pltpu.VMEM((ZB, D), jnp.float32),
        pltpu.VMEM_SHARED((NPAD, D), jnp.float32),
        pltpu.SemaphoreType.DMA,
    ],
)

_sc_cnt = pl.kernel(
    _sc_cnt_body,
    out_type=(jax.ShapeDtypeStruct((NC, NPAD, D), jnp.float32),),
    mesh=_sc_mesh,
    scratch_types=[
        pltpu.VMEM((NCHUNK, CK), jnp.int32),
        pltpu.VMEM((ZB, D), jnp.float32),
        pltpu.VMEM_SHARED((NPAD, D), jnp.float32),
        pltpu.SemaphoreType.DMA,
    ],
)

RB = 1000


def _tc_layer1_body(sums_ref, cnt_ref, x_ref, wl_ref, bl_ref, wr_ref, o_ref):
    ssum = sums_ref[0] + sums_ref[1]
    cnt = cnt_ref[0] + cnt_ref[1]
    mean = ssum / jnp.maximum(cnt, 1.0)
    h = (jnp.dot(mean, wl_ref[...], preferred_element_type=jnp.float32,
                 precision=lax.Precision.HIGHEST)
         + jnp.dot(x_ref[...], wr_ref[...], preferred_element_type=jnp.float32,
                   precision=lax.Precision.HIGHEST)
         + bl_ref[...])
    o_ref[...] = jnp.maximum(h, 0.0)


def _tc_layer2_body(sums_ref, cnt_ref, h_ref, wl_ref, bl_ref, wr_ref,
                    wo_ref, bo_ref, o_ref):
    ssum = sums_ref[0] + sums_ref[1]
    cnt = cnt_ref[0] + cnt_ref[1]
    mean = ssum / jnp.maximum(cnt, 1.0)
    h = (jnp.dot(mean, wl_ref[...], preferred_element_type=jnp.float32,
                 precision=lax.Precision.HIGHEST)
         + jnp.dot(h_ref[...], wr_ref[...], preferred_element_type=jnp.float32,
                   precision=lax.Precision.HIGHEST)
         + bl_ref[...])
    h = jnp.maximum(h, 0.0)
    o_ref[...] = (jnp.dot(h, wo_ref[...], preferred_element_type=jnp.float32,
                          precision=lax.Precision.HIGHEST)
                  + bo_ref[...])


def _tc_layer1(sums, cnt, x, Wl, bl, Wr):
    grid = (N // RB,)
    return pl.pallas_call(
        _tc_layer1_body,
        grid=grid,
        in_specs=[
            pl.BlockSpec((NC, RB, D), lambda i: (0, i, 0)),
            pl.BlockSpec((NC, RB, D), lambda i: (0, i, 0)),
            pl.BlockSpec((RB, D), lambda i: (i, 0)),
            pl.BlockSpec((D, H), lambda i: (0, 0)),
            pl.BlockSpec((1, H), lambda i: (0, 0)),
            pl.BlockSpec((D, H), lambda i: (0, 0)),
        ],
        out_specs=pl.BlockSpec((RB, H), lambda i: (i, 0)),
        out_shape=jax.ShapeDtypeStruct((N, H), jnp.float32),
    )(sums, cnt, x, Wl, bl, Wr)


def _tc_layer2(sums, cnt, h, Wl, bl, Wr, Wout, bout):
    grid = (N // RB,)
    return pl.pallas_call(
        _tc_layer2_body,
        grid=grid,
        in_specs=[
            pl.BlockSpec((NC, RB, H), lambda i: (0, i, 0)),
            pl.BlockSpec((NC, RB, D), lambda i: (0, i, 0)),
            pl.BlockSpec((RB, H), lambda i: (i, 0)),
            pl.BlockSpec((H, H), lambda i: (0, 0)),
            pl.BlockSpec((1, H), lambda i: (0, 0)),
            pl.BlockSpec((H, H), lambda i: (0, 0)),
            pl.BlockSpec((H, C), lambda i: (0, 0)),
            pl.BlockSpec((1, C), lambda i: (0, 0)),
        ],
        out_specs=pl.BlockSpec((RB, C), lambda i: (i, 0)),
        out_shape=jax.ShapeDtypeStruct((N, C), jnp.float32),
    )(sums, cnt, h, Wl, bl, Wr, Wout, bout)


def kernel(x, edge_index, Wl1, bl1, Wr1, Wl2, bl2, Wr2, Wout, bout):
    src3 = edge_index[0].reshape(NW, NCHUNK, CK)
    dst3 = edge_index[1].reshape(NW, NCHUNK, CK)

    (cnt,) = _sc_cnt(dst3)
    (sums1,) = _sc_agg(x, src3, dst3)
    h1 = _tc_layer1(sums1, cnt, x, Wl1, bl1.reshape(1, H), Wr1)
    (sums2,) = _sc_agg(h1, src3, dst3)
    out = _tc_layer2(sums2, cnt, h1, Wl2, bl2.reshape(1, H), Wr2,
                     Wout, bout.reshape(1, C))
    return out
